# Initial kernel scaffold; baseline (speedup 1.0000x reference)
import jax
import jax.numpy as jnp
from jax.experimental import pallas as pl


def kernel(stu_emb, t1_prob, t2_prob, classifier_weight, labels, weights_param):
    raise NotImplementedError("write your pallas kernel here")


# speedup vs baseline: 7.6424x; 7.6424x over previous
import functools

import jax
import jax.numpy as jnp
from jax.experimental import pallas as pl

B = 4096
D = 768
C1 = 40
C2 = 40
NC = C1 + C2
MARGIN = 1.0
EPS = 1e-6
BB = 512
NB = B // BB


def _rank_weights(p, w3):
    pk = p[:, :, None]
    pj = p[:, None, :]
    ik = jax.lax.broadcasted_iota(jnp.int32, (1, C1, C1), 1)
    ij = jax.lax.broadcasted_iota(jnp.int32, (1, C1, C1), 2)
    ahead = (pk > pj) | ((pk == pj) & (ik < ij))
    rank = jnp.sum(ahead.astype(jnp.int32), axis=1)
    onehot = rank[:, :, None] == jax.lax.broadcasted_iota(jnp.int32, (1, 1, C1), 2)
    return jnp.sum(jnp.where(onehot, w3, 0.0), axis=2)


def _loss_kernel(stu_ref, p1_ref, p2_ref, cw_ref, lab_ref, w_ref, out_ref):
    stu = stu_ref[...] + EPS
    cw = cw_ref[...]

    dots = jax.lax.dot_general(
        stu, cw, (((1,), (1,)), ((), ())),
        preferred_element_type=jnp.float32)
    u2 = jnp.sum(stu * stu, axis=1, keepdims=True)
    v2 = jnp.sum(cw * cw, axis=1, keepdims=True)
    dn2 = u2 - 2.0 * dots + v2.reshape(1, NC)
    dn = jnp.sqrt(jnp.maximum(dn2, 0.0))

    labels = lab_ref[...]
    col = jax.lax.broadcasted_iota(jnp.int32, (1, NC), 1)
    dp = jnp.sum(jnp.where(labels == col, dn, 0.0), axis=1, keepdims=True)

    hinge = jnp.maximum(dp - dn + MARGIN, 0.0)

    w = jax.nn.sigmoid(w_ref[...])
    wi = jax.lax.broadcasted_iota(jnp.int32, (1, C1), 1)
    w3 = jnp.where(wi == 0, 0.0, w).reshape(1, 1, C1)

    wsel1 = _rank_weights(p1_ref[...], w3)
    wsel2 = _rank_weights(p2_ref[...], w3)

    block_sum = (jnp.sum(wsel1 * hinge[:, :C1]) +
                 jnp.sum(wsel2 * hinge[:, C1:])).reshape(1, 1)

    @pl.when(pl.program_id(0) == 0)
    def _init():
        out_ref[...] = jnp.zeros((1, 1), jnp.float32)

    out_ref[...] += block_sum

    @pl.when(pl.program_id(0) == NB - 1)
    def _finish():
        out_ref[...] = out_ref[...] * (1.0 / B)


@functools.partial(jax.jit, static_argnames=("interpret",))
def kernel(stu_emb, t1_prob, t2_prob, classifier_weight, labels, weights_param,
           interpret=False):
    out = pl.pallas_call(
        _loss_kernel,
        grid=(NB,),
        in_specs=[
            pl.BlockSpec((BB, D), lambda i: (i, 0)),
            pl.BlockSpec((BB, C1), lambda i: (i, 0)),
            pl.BlockSpec((BB, C2), lambda i: (i, 0)),
            pl.BlockSpec((NC, D), lambda i: (0, 0)),
            pl.BlockSpec((BB, 1), lambda i: (i, 0)),
            pl.BlockSpec((1, C1), lambda i: (0, 0)),
        ],
        out_specs=pl.BlockSpec((1, 1), lambda i: (0, 0)),
        out_shape=jax.ShapeDtypeStruct((1, 1), jnp.float32),
        interpret=interpret,
    )(
        stu_emb,
        t1_prob,
        t2_prob,
        classifier_weight,
        labels.astype(jnp.int32).reshape(B, 1),
        weights_param.reshape(1, C1),
    )
    return out.reshape(())


# speedup vs baseline: 33.6315x; 4.4006x over previous
import functools

import jax
import jax.numpy as jnp
from jax.experimental import pallas as pl
from jax.experimental.pallas import tpu as pltpu

B = 4096
D = 768
C1 = 40
C2 = 40
NC = C1 + C2
MARGIN = 1.0
EPS = 1e-6
BB = 512
NB = B // BB


def _rank_weights(pT, wsig):
    pk = pT[:, None, :]
    pj = pT[None, :, :]
    ik = jax.lax.broadcasted_iota(jnp.int32, (C1, C1, 1), 0)
    ij = jax.lax.broadcasted_iota(jnp.int32, (C1, C1, 1), 1)
    ahead = (pk > pj) | ((pk == pj) & (ik < ij))
    rank = jnp.sum(ahead.astype(jnp.int32), axis=0)
    wsel = jnp.zeros(pT.shape, jnp.float32)
    for r in range(1, C1):
        wsel = wsel + jnp.where(rank == r, wsig[r], 0.0)
    return wsel


def _loss_kernel(stuT_ref, p1T_ref, p2T_ref, cw_ref, labT_ref, w_ref, out_ref):
    ustar = stuT_ref[...] + EPS
    cw = cw_ref[...]

    dots = jax.lax.dot_general(
        cw, ustar, (((1,), (0,)), ((), ())),
        preferred_element_type=jnp.float32)
    u2 = jnp.sum(ustar * ustar, axis=0, keepdims=True)
    v2 = jnp.sum(cw * cw, axis=1, keepdims=True)
    dn = jnp.sqrt(jnp.maximum(u2 - 2.0 * dots + v2, 0.0))

    lab = labT_ref[...]
    row = jax.lax.broadcasted_iota(jnp.int32, (NC, 1), 0)
    dp = jnp.sum(jnp.where(row == lab, dn, 0.0), axis=0, keepdims=True)

    hinge = jnp.maximum(dp - dn + MARGIN, 0.0)

    wsig = [jax.nn.sigmoid(w_ref[0, r]) for r in range(C1)]
    wsel1 = _rank_weights(p1T_ref[...], wsig)
    wsel2 = _rank_weights(p2T_ref[...], wsig)

    block_sum = jnp.sum(wsel1 * hinge[:C1, :] +
                        wsel2 * hinge[C1:, :]).reshape(1, 1)

    @pl.when(pl.program_id(0) == 0)
    def _init():
        out_ref[...] = jnp.zeros((1, 1), jnp.float32)

    out_ref[...] += block_sum

    @pl.when(pl.program_id(0) == NB - 1)
    def _finish():
        out_ref[...] = out_ref[...] * (1.0 / B)


@functools.partial(jax.jit, static_argnames=("interpret",))
def kernel(stu_emb, t1_prob, t2_prob, classifier_weight, labels, weights_param,
           interpret=False):
    out = pl.pallas_call(
        _loss_kernel,
        grid=(NB,),
        in_specs=[
            pl.BlockSpec((D, BB), lambda i: (0, i)),
            pl.BlockSpec((C1, BB), lambda i: (0, i)),
            pl.BlockSpec((C2, BB), lambda i: (0, i)),
            pl.BlockSpec((NC, D), lambda i: (0, 0)),
            pl.BlockSpec((1, BB), lambda i: (0, i)),
            pl.BlockSpec(memory_space=pltpu.SMEM),
        ],
        out_specs=pl.BlockSpec((1, 1), lambda i: (0, 0)),
        out_shape=jax.ShapeDtypeStruct((1, 1), jnp.float32),
        interpret=interpret,
    )(
        stu_emb.T,
        t1_prob.T,
        t2_prob.T,
        classifier_weight,
        labels.astype(jnp.int32).reshape(1, B),
        weights_param.reshape(1, C1),
    )
    return out.reshape(())


# speedup vs baseline: 36.7594x; 1.0930x over previous
import functools

import jax
import jax.numpy as jnp
from jax.experimental import pallas as pl
from jax.experimental.pallas import tpu as pltpu

B = 4096
D = 768
C1 = 40
C2 = 40
NC = C1 + C2
MARGIN = 1.0
EPS = 1e-6
BB = 512
NB = B // BB


def _rank_weights(pT, wsig):
    CK = 8
    pj = pT[None, :, :]
    ij = jax.lax.broadcasted_iota(jnp.int32, (1, C1, 1), 1)
    rank = jnp.zeros(pT.shape, jnp.float32)
    for c in range(0, C1, CK):
        pk = pT[c:c + CK, None, :]
        ik = c + jax.lax.broadcasted_iota(jnp.int32, (CK, 1, 1), 0)
        ge = jnp.where(pk >= pj, 1.0, 0.0)
        gt = jnp.where(pk > pj, 1.0, 0.0)
        ahead = jnp.where(ik < ij, ge, gt)
        rank = rank + jnp.sum(ahead, axis=0)
    wsel = jnp.zeros(pT.shape, jnp.float32)
    for r in range(1, C1):
        wsel = wsel + jnp.where(rank == float(r), wsig[r], 0.0)
    return wsel


def _loss_kernel(stuT_ref, p1T_ref, p2T_ref, cw_ref, labT_ref, w_ref, out_ref):
    ustar = stuT_ref[...] + EPS
    cw = cw_ref[...]

    dots = jax.lax.dot_general(
        cw, ustar, (((1,), (0,)), ((), ())),
        preferred_element_type=jnp.float32)
    u2 = jnp.sum(ustar * ustar, axis=0, keepdims=True)
    v2 = jnp.sum(cw * cw, axis=1, keepdims=True)
    dn = jnp.sqrt(jnp.maximum(u2 - 2.0 * dots + v2, 0.0))

    lab = labT_ref[...]
    row = jax.lax.broadcasted_iota(jnp.int32, (NC, 1), 0)
    dp = jnp.sum(jnp.where(row == lab, dn, 0.0), axis=0, keepdims=True)

    hinge = jnp.maximum(dp - dn + MARGIN, 0.0)

    wsig = [jax.nn.sigmoid(w_ref[0, r]) for r in range(C1)]
    wsel1 = _rank_weights(p1T_ref[...], wsig)
    wsel2 = _rank_weights(p2T_ref[...], wsig)

    block_sum = jnp.sum(wsel1 * hinge[:C1, :] +
                        wsel2 * hinge[C1:, :]).reshape(1, 1)

    @pl.when(pl.program_id(0) == 0)
    def _init():
        out_ref[...] = jnp.zeros((1, 1), jnp.float32)

    out_ref[...] += block_sum

    @pl.when(pl.program_id(0) == NB - 1)
    def _finish():
        out_ref[...] = out_ref[...] * (1.0 / B)


@functools.partial(jax.jit, static_argnames=("interpret",))
def kernel(stu_emb, t1_prob, t2_prob, classifier_weight, labels, weights_param,
           interpret=False):
    out = pl.pallas_call(
        _loss_kernel,
        grid=(NB,),
        in_specs=[
            pl.BlockSpec((D, BB), lambda i: (0, i)),
            pl.BlockSpec((C1, BB), lambda i: (0, i)),
            pl.BlockSpec((C2, BB), lambda i: (0, i)),
            pl.BlockSpec((NC, D), lambda i: (0, 0)),
            pl.BlockSpec((1, BB), lambda i: (0, i)),
            pl.BlockSpec(memory_space=pltpu.SMEM),
        ],
        out_specs=pl.BlockSpec((1, 1), lambda i: (0, 0)),
        out_shape=jax.ShapeDtypeStruct((1, 1), jnp.float32),
        interpret=interpret,
    )(
        stu_emb.T,
        t1_prob.T,
        t2_prob.T,
        classifier_weight,
        labels.astype(jnp.int32).reshape(1, B),
        weights_param.reshape(1, C1),
    )
    return out.reshape(())


# speedup vs baseline: 37.4332x; 1.0183x over previous
import functools

import jax
import jax.numpy as jnp
from jax.experimental import pallas as pl
from jax.experimental.pallas import tpu as pltpu

B = 4096
D = 768
C1 = 40
C2 = 40
NC = C1 + C2
MARGIN = 1.0
EPS = 1e-6
BB = 512
NB = B // BB


def _rank_weights(pT, wsig):
    CK = 8
    pj = pT[None, :, :]
    ij = jax.lax.broadcasted_iota(jnp.int32, (1, C1, 1), 1)
    rank = jnp.zeros(pT.shape, jnp.float32)
    for c in range(0, C1, CK):
        pk = pT[c:c + CK, None, :]
        ik = c + jax.lax.broadcasted_iota(jnp.int32, (CK, 1, 1), 0)
        ge = jnp.where(pk >= pj, 1.0, 0.0)
        gt = jnp.where(pk > pj, 1.0, 0.0)
        ahead = jnp.where(ik < ij, ge, gt)
        rank = rank + jnp.sum(ahead, axis=0)
    ri = rank.astype(jnp.int32)
    bit = [(ri & (1 << k)) != 0 for k in range(4)]
    t = [jnp.where(bit[0], wsig[2 * i + 1], wsig[2 * i]) for i in range(20)]
    t = [jnp.where(bit[1], t[2 * i + 1], t[2 * i]) for i in range(10)]
    t = [jnp.where(bit[2], t[2 * i + 1], t[2 * i]) for i in range(5)]
    u = [jnp.where(bit[3], t[1], t[0]), jnp.where(bit[3], t[3], t[2]), t[4]]
    return jnp.where(ri >= 32, u[2], jnp.where(ri >= 16, u[1], u[0]))


def _loss_kernel(stuT_ref, p1T_ref, p2T_ref, cw_ref, labT_ref, w_ref, out_ref):
    ustar = stuT_ref[...] + EPS
    cw = cw_ref[...]

    dots = jax.lax.dot_general(
        cw, ustar, (((1,), (0,)), ((), ())),
        preferred_element_type=jnp.float32)
    u2 = jnp.sum(ustar * ustar, axis=0, keepdims=True)
    v2 = jnp.sum(cw * cw, axis=1, keepdims=True)
    dn = jnp.sqrt(jnp.maximum(u2 - 2.0 * dots + v2, 0.0))

    lab = labT_ref[...]
    row = jax.lax.broadcasted_iota(jnp.int32, (NC, 1), 0)
    dp = jnp.sum(jnp.where(row == lab, dn, 0.0), axis=0, keepdims=True)

    hinge = jnp.maximum(dp - dn + MARGIN, 0.0)

    wsig = [jnp.float32(0.0)] + [jax.nn.sigmoid(w_ref[0, r]) for r in range(1, C1)]
    wsel1 = _rank_weights(p1T_ref[...], wsig)
    wsel2 = _rank_weights(p2T_ref[...], wsig)

    block_sum = jnp.sum(wsel1 * hinge[:C1, :] +
                        wsel2 * hinge[C1:, :]).reshape(1, 1)

    @pl.when(pl.program_id(0) == 0)
    def _init():
        out_ref[...] = jnp.zeros((1, 1), jnp.float32)

    out_ref[...] += block_sum

    @pl.when(pl.program_id(0) == NB - 1)
    def _finish():
        out_ref[...] = out_ref[...] * (1.0 / B)


@functools.partial(jax.jit, static_argnames=("interpret",))
def kernel(stu_emb, t1_prob, t2_prob, classifier_weight, labels, weights_param,
           interpret=False):
    out = pl.pallas_call(
        _loss_kernel,
        grid=(NB,),
        in_specs=[
            pl.BlockSpec((D, BB), lambda i: (0, i)),
            pl.BlockSpec((C1, BB), lambda i: (0, i)),
            pl.BlockSpec((C2, BB), lambda i: (0, i)),
            pl.BlockSpec((NC, D), lambda i: (0, 0)),
            pl.BlockSpec((1, BB), lambda i: (0, i)),
            pl.BlockSpec(memory_space=pltpu.SMEM),
        ],
        out_specs=pl.BlockSpec((1, 1), lambda i: (0, 0)),
        out_shape=jax.ShapeDtypeStruct((1, 1), jnp.float32),
        interpret=interpret,
    )(
        stu_emb.T,
        t1_prob.T,
        t2_prob.T,
        classifier_weight,
        labels.astype(jnp.int32).reshape(1, B),
        weights_param.reshape(1, C1),
    )
    return out.reshape(())


# speedup vs baseline: 87.8643x; 2.3472x over previous
import functools

import jax
import jax.numpy as jnp
from jax.experimental import pallas as pl
from jax.experimental.pallas import tpu as pltpu

B = 4096
D = 768
C1 = 40
C2 = 40
NC = C1 + C2
MARGIN = 1.0
EPS = 1e-6
BB = 512
NB = B // BB


def _rank_weights(pT, wsig):
    CK = 8
    pj = pT[None, :, :]
    ij = jax.lax.broadcasted_iota(jnp.int32, (1, C1, 1), 1)
    rank = jnp.zeros(pT.shape, jnp.float32)
    for c in range(0, C1, CK):
        pk = pT[c:c + CK, None, :]
        ik = c + jax.lax.broadcasted_iota(jnp.int32, (CK, 1, 1), 0)
        ge = jnp.where(pk >= pj, 1.0, 0.0)
        gt = jnp.where(pk > pj, 1.0, 0.0)
        ahead = jnp.where(ik < ij, ge, gt)
        rank = rank + jnp.sum(ahead, axis=0)
    ri = rank.astype(jnp.int32)
    bit = [(ri & (1 << k)) != 0 for k in range(4)]
    t = [jnp.where(bit[0], wsig[2 * i + 1], wsig[2 * i]) for i in range(20)]
    t = [jnp.where(bit[1], t[2 * i + 1], t[2 * i]) for i in range(10)]
    t = [jnp.where(bit[2], t[2 * i + 1], t[2 * i]) for i in range(5)]
    u = [jnp.where(bit[3], t[1], t[0]), jnp.where(bit[3], t[3], t[2]), t[4]]
    return jnp.where(ri >= 32, u[2], jnp.where(ri >= 16, u[1], u[0]))


def _loss_kernel(stu_ref, p1T_ref, p2T_ref, cw_ref, labT_ref, w_ref, out_ref):
    ustar = stu_ref[...] + EPS
    cw = cw_ref[...]

    dots = jax.lax.dot_general(
        cw, ustar, (((1,), (1,)), ((), ())),
        preferred_element_type=jnp.float32)
    u2 = jax.lax.dot_general(
        jnp.ones((1, D), jnp.float32), ustar * ustar, (((1,), (1,)), ((), ())),
        preferred_element_type=jnp.float32)
    v2 = jnp.sum(cw * cw, axis=1, keepdims=True)
    dn = jnp.sqrt(jnp.maximum(u2 - 2.0 * dots + v2, 0.0))

    lab = labT_ref[...]
    row = jax.lax.broadcasted_iota(jnp.int32, (NC, 1), 0)
    dp = jnp.sum(jnp.where(row == lab, dn, 0.0), axis=0, keepdims=True)

    hinge = jnp.maximum(dp - dn + MARGIN, 0.0)

    wsig = [jnp.float32(0.0)] + [jax.nn.sigmoid(w_ref[0, r]) for r in range(1, C1)]
    wsel1 = _rank_weights(p1T_ref[...], wsig)
    wsel2 = _rank_weights(p2T_ref[...], wsig)

    block_sum = jnp.sum(wsel1 * hinge[:C1, :] +
                        wsel2 * hinge[C1:, :]).reshape(1, 1)

    @pl.when(pl.program_id(0) == 0)
    def _init():
        out_ref[...] = jnp.zeros((1, 1), jnp.float32)

    out_ref[...] += block_sum

    @pl.when(pl.program_id(0) == NB - 1)
    def _finish():
        out_ref[...] = out_ref[...] * (1.0 / B)


@functools.partial(jax.jit, static_argnames=("interpret",))
def kernel(stu_emb, t1_prob, t2_prob, classifier_weight, labels, weights_param,
           interpret=False):
    out = pl.pallas_call(
        _loss_kernel,
        grid=(NB,),
        in_specs=[
            pl.BlockSpec((BB, D), lambda i: (i, 0)),
            pl.BlockSpec((C1, BB), lambda i: (0, i)),
            pl.BlockSpec((C2, BB), lambda i: (0, i)),
            pl.BlockSpec((NC, D), lambda i: (0, 0)),
            pl.BlockSpec((1, BB), lambda i: (0, i)),
            pl.BlockSpec(memory_space=pltpu.SMEM),
        ],
        out_specs=pl.BlockSpec((1, 1), lambda i: (0, 0)),
        out_shape=jax.ShapeDtypeStruct((1, 1), jnp.float32),
        interpret=interpret,
    )(
        stu_emb,
        t1_prob.T,
        t2_prob.T,
        classifier_weight,
        labels.astype(jnp.int32).reshape(1, B),
        weights_param.reshape(1, C1),
    )
    return out.reshape(())
